# baseline (speedup 1.0000x reference)
import functools
import math

import jax
import jax.numpy as jnp
from jax import lax
from jax.experimental import pallas as pl
from jax.experimental.pallas import tpu as pltpu
from jax.experimental.pallas import tpu_sc as plsc

VOCAB_SIZE = 1000000
EMB = 128
SEQ = 2048
BATCH = 4

NC, NS, L = 2, 16, 16
NW = NC * NS
ROWS = SEQ * BATCH
RPW = ROWS // NW
SPW = SEQ // NW
IDX_CHUNK = 128
N_CHUNKS = RPW // IDX_CHUNK
SPC = SPW // N_CHUNKS
SCALE = math.sqrt(EMB)


def _positions():
    denom = jnp.exp(-1.0 * jnp.arange(0, EMB, 2, dtype=jnp.float32)
                    * math.log(10000.0) / EMB)
    pos = jnp.arange(0, SEQ, dtype=jnp.float32).reshape(SEQ, 1)
    p = jnp.zeros((SEQ, EMB), dtype=jnp.float32)
    p = p.at[:, 0::2].set(jnp.sin(pos * denom))
    p = p.at[:, 1::2].set(jnp.cos(pos * denom))
    return p


@functools.partial(
    pl.kernel,
    mesh=plsc.VectorSubcoreMesh(core_axis_name="c", subcore_axis_name="s"),
    out_type=jax.ShapeDtypeStruct((NW, RPW, EMB), jnp.float32),
    scratch_types=[
        pltpu.VMEM((N_CHUNKS, IDX_CHUNK), jnp.int32),
        pltpu.VMEM((RPW, EMB), jnp.float32),
        pltpu.VMEM((SPW, EMB), jnp.float32),
        pltpu.SemaphoreType.DMA((N_CHUNKS,)),
        pltpu.SemaphoreType.DMA,
        pltpu.SemaphoreType.DMA,
    ],
)
def _sc_embed(idx_hbm, pos_hbm, table_hbm, out_hbm, idx_v, rows_v, pos_v,
              gsem, wsem, psem):
    wid = lax.axis_index("s") * NC + lax.axis_index("c")
    pos_cp = pltpu.async_copy(pos_hbm.at[wid], pos_v, psem)
    pltpu.sync_copy(idx_hbm.at[wid], idx_v)
    gathers = [
        pltpu.async_copy(
            table_hbm.at[idx_v.at[c]],
            rows_v.at[pl.ds(c * IDX_CHUNK, IDX_CHUNK)],
            gsem.at[c],
        )
        for c in range(N_CHUNKS)
    ]
    pos_cp.wait()

    writes = []
    for c in range(N_CHUNKS):
        gathers[c].wait()

        @plsc.parallel_loop(0, SPC, unroll=8)
        def _body(s, c=c):
            for j in range(EMB // L):
                pv = pos_v[c * SPC + s, pl.ds(j * L, L)]
                for b in range(BATCH):
                    sl = (c * IDX_CHUNK + s * BATCH + b, pl.ds(j * L, L))
                    rows_v[sl] = rows_v[sl] * SCALE + pv

        writes.append(
            pltpu.async_copy(
                rows_v.at[pl.ds(c * IDX_CHUNK, IDX_CHUNK)],
                out_hbm.at[wid, pl.ds(c * IDX_CHUNK, IDX_CHUNK)],
                wsem,
            )
        )
    for wcp in writes:
        wcp.wait()


def kernel(x, table):
    idx = x.astype(jnp.int32).reshape(NW, N_CHUNKS, IDX_CHUNK)
    pos = _positions().reshape(NW, SPW, EMB)
    out = _sc_embed(idx, pos, table)
    return out.reshape(SEQ, BATCH, EMB)


# speedup vs baseline: 1.0221x; 1.0221x over previous
import functools
import math

import jax
import jax.numpy as jnp
from jax import lax
from jax.experimental import pallas as pl
from jax.experimental.pallas import tpu as pltpu
from jax.experimental.pallas import tpu_sc as plsc

VOCAB_SIZE = 1000000
EMB = 128
SEQ = 2048
BATCH = 4

NC, NS, L = 2, 16, 16
NW = NC * NS
ROWS = SEQ * BATCH
RPW = ROWS // NW
SPW = SEQ // NW
IDX_CHUNK = 128
N_CHUNKS = RPW // IDX_CHUNK
SPC = SPW // N_CHUNKS
SCALE = math.sqrt(EMB)


def _positions():
    denom = jnp.exp(-1.0 * jnp.arange(0, EMB, 2, dtype=jnp.float32)
                    * math.log(10000.0) / EMB)
    pos = jnp.arange(0, SEQ, dtype=jnp.float32).reshape(SEQ, 1)
    p = jnp.zeros((SEQ, EMB), dtype=jnp.float32)
    p = p.at[:, 0::2].set(jnp.sin(pos * denom))
    p = p.at[:, 1::2].set(jnp.cos(pos * denom))
    return p


@functools.partial(
    pl.kernel,
    mesh=plsc.VectorSubcoreMesh(core_axis_name="c", subcore_axis_name="s"),
    out_type=jax.ShapeDtypeStruct((NW, RPW, EMB), jnp.float32),
    scratch_types=[
        pltpu.VMEM((N_CHUNKS, IDX_CHUNK), jnp.int32),
        pltpu.VMEM((RPW, EMB), jnp.float32),
        pltpu.VMEM((SPW, EMB), jnp.float32),
        pltpu.SemaphoreType.DMA((N_CHUNKS,)),
        pltpu.SemaphoreType.DMA,
        pltpu.SemaphoreType.DMA,
    ],
)
def _sc_embed(idx_hbm, pos_hbm, table_hbm, out_hbm, idx_v, rows_v, pos_v,
              gsem, wsem, psem):
    wid = lax.axis_index("s") * NC + lax.axis_index("c")
    pos_cp = pltpu.async_copy(pos_hbm.at[wid], pos_v, psem)
    pltpu.sync_copy(idx_hbm.at[wid], idx_v)
    gathers = [
        pltpu.async_copy(
            table_hbm.at[idx_v.at[c]],
            rows_v.at[pl.ds(c * IDX_CHUNK, IDX_CHUNK)],
            gsem.at[c],
        )
        for c in range(N_CHUNKS)
    ]
    pos_cp.wait()

    writes = []
    for c in range(N_CHUNKS):
        gathers[c].wait()

        @plsc.parallel_loop(0, SPC, unroll=4)
        def _body(s, c=c):
            for j in range(EMB // L):
                pv = pos_v[c * SPC + s, pl.ds(j * L, L)]
                for b in range(BATCH):
                    sl = (c * IDX_CHUNK + s * BATCH + b, pl.ds(j * L, L))
                    rows_v[sl] = rows_v[sl] * SCALE + pv

        writes.append(
            pltpu.async_copy(
                rows_v.at[pl.ds(c * IDX_CHUNK, IDX_CHUNK)],
                out_hbm.at[wid, pl.ds(c * IDX_CHUNK, IDX_CHUNK)],
                wsem,
            )
        )
    for wcp in writes:
        wcp.wait()


def kernel(x, table):
    idx = x.astype(jnp.int32).reshape(NW, N_CHUNKS, IDX_CHUNK)
    pos = _positions().reshape(NW, SPW, EMB)
    out = _sc_embed(idx, pos, table)
    return out.reshape(SEQ, BATCH, EMB)
